# baseline (speedup 1.0000x reference)
import jax
import jax.numpy as jnp
from jax import lax
from jax.experimental import pallas as pl
from jax.experimental.pallas import tpu as pltpu
from jax.experimental.pallas import tpu_sc as plsc

NUM_CLASSES = 256
GRID = 64
N = GRID * GRID * GRID
NUM_CORES = 2
NUM_SUBCORES = 16
NUM_WORKERS = NUM_CORES * NUM_SUBCORES
LANES = 16

CLASS_GROUPS = 8
SPATIAL_GROUPS = NUM_WORKERS // CLASS_GROUPS
CC = NUM_CLASSES // CLASS_GROUPS
SPAN = N // SPATIAL_GROUPS
CHUNK = 1024
STEPS = SPAN // CHUNK
NBUF = 2


def _scatter_pass(vox_v, tile_v, row_base, value16, iota16):
    for k in range(CHUNK // LANES):
        vox16 = vox_v[pl.ds(k * LANES, LANES)]
        rows = vox16 - row_base
        mask = (vox16 >= row_base) & (vox16 < row_base + CC)
        cols = iota16 + (k * LANES)
        plsc.store_scatter(tile_v, [rows, cols], value16, mask=mask)


def _onehot_body(vox_hbm, out_hbm, vox0, vox1, tile0, tile1, sem0, sem1):
    cid = lax.axis_index("c")
    sid = lax.axis_index("s")
    wid = sid * NUM_CORES + cid
    cg = wid % CLASS_GROUPS
    sg = wid // CLASS_GROUPS
    row_base = cg * CC
    col_base = sg * SPAN

    vox_bufs = (vox0, vox1)
    tile_bufs = (tile0, tile1)
    sems = (sem0, sem1)

    zeros16 = jnp.zeros((LANES,), jnp.float32)
    ones16 = jnp.full((LANES,), 1.0, jnp.float32)
    iota16 = lax.iota(jnp.int32, LANES)

    def _zero_row(r, _):
        for b in range(NBUF):
            for k in range(CHUNK // LANES):
                tile_bufs[b][r, pl.ds(k * LANES, LANES)] = zeros16
        return 0

    lax.fori_loop(0, CC, _zero_row, 0)

    def _out_slice(j):
        off = pl.multiple_of(col_base + j * CHUNK, CHUNK)
        return out_hbm.at[pl.ds(row_base, CC), pl.ds(off, CHUNK)]

    def _pair(t, _):
        for b in range(NBUF):
            j = t * NBUF + b

            @pl.when(t >= 1)
            def _drain():
                pltpu.make_async_copy(tile_bufs[b], _out_slice(j - NBUF), sems[b]).wait()
                _scatter_pass(vox_bufs[b], tile_bufs[b], row_base, zeros16, iota16)

            off = pl.multiple_of(col_base + j * CHUNK, CHUNK)
            pltpu.sync_copy(vox_hbm.at[pl.ds(off, CHUNK)], vox_bufs[b])
            _scatter_pass(vox_bufs[b], tile_bufs[b], row_base, ones16, iota16)
            pltpu.async_copy(tile_bufs[b], _out_slice(j), sems[b])
        return 0

    lax.fori_loop(0, STEPS // NBUF, _pair, 0)

    for b in range(NBUF):
        pltpu.make_async_copy(tile_bufs[b], _out_slice(STEPS - NBUF + b), sems[b]).wait()


def kernel(voxels):
    vox = voxels.reshape(N).astype(jnp.int32)
    mesh = plsc.VectorSubcoreMesh(
        core_axis_name="c",
        subcore_axis_name="s",
        num_cores=NUM_CORES,
        num_subcores=NUM_SUBCORES,
    )
    out = pl.kernel(
        _onehot_body,
        out_type=jax.ShapeDtypeStruct((NUM_CLASSES, N), jnp.float32),
        mesh=mesh,
        scratch_types=[
            pltpu.VMEM((CHUNK,), jnp.int32),
            pltpu.VMEM((CHUNK,), jnp.int32),
            pltpu.VMEM((CC, CHUNK), jnp.float32),
            pltpu.VMEM((CC, CHUNK), jnp.float32),
            pltpu.SemaphoreType.DMA,
            pltpu.SemaphoreType.DMA,
        ],
        compiler_params=pltpu.CompilerParams(
            use_tc_tiling_on_sc=False, needs_layout_passes=False
        ),
    )(vox)
    return out.reshape(NUM_CLASSES, GRID, GRID, GRID)


# speedup vs baseline: 1.6238x; 1.6238x over previous
import jax
import jax.numpy as jnp
from jax import lax
from jax.experimental import pallas as pl
from jax.experimental.pallas import tpu as pltpu
from jax.experimental.pallas import tpu_sc as plsc

NUM_CLASSES = 256
GRID = 64
N = GRID * GRID * GRID
NUM_CORES = 2
NUM_SUBCORES = 16
NUM_WORKERS = NUM_CORES * NUM_SUBCORES
LANES = 16

PER_WORKER = N // NUM_WORKERS
CHUNK = 128
STEPS = PER_WORKER // CHUNK
NBUF = 2


def _scatter_pass(vox_v, tile_v, value16, iota16):
    for k in range(CHUNK // LANES):
        vox16 = vox_v[pl.ds(k * LANES, LANES)]
        rows = iota16 + (k * LANES)
        plsc.store_scatter(tile_v, [rows, vox16], value16)


def _onehot_body(vox_hbm, out_hbm, vox0, vox1, tile0, tile1, sem0, sem1):
    cid = lax.axis_index("c")
    sid = lax.axis_index("s")
    wid = sid * NUM_CORES + cid
    row_base = wid * PER_WORKER

    vox_bufs = (vox0, vox1)
    tile_bufs = (tile0, tile1)
    sems = (sem0, sem1)

    zeros16 = jnp.zeros((LANES,), jnp.float32)
    ones16 = jnp.full((LANES,), 1.0, jnp.float32)
    iota16 = lax.iota(jnp.int32, LANES)

    def _zero_row(r, _):
        for b in range(NBUF):
            for k in range(NUM_CLASSES // LANES):
                tile_bufs[b][r, pl.ds(k * LANES, LANES)] = zeros16
        return 0

    lax.fori_loop(0, CHUNK, _zero_row, 0)

    def _out_slice(j):
        off = pl.multiple_of(row_base + j * CHUNK, CHUNK)
        return out_hbm.at[pl.ds(off, CHUNK), :]

    def _pair(t, _):
        for b in range(NBUF):
            j = t * NBUF + b

            @pl.when(t >= 1)
            def _drain():
                pltpu.make_async_copy(tile_bufs[b], _out_slice(j - NBUF), sems[b]).wait()
                _scatter_pass(vox_bufs[b], tile_bufs[b], zeros16, iota16)

            off = pl.multiple_of(row_base + j * CHUNK, CHUNK)
            pltpu.sync_copy(vox_hbm.at[pl.ds(off, CHUNK)], vox_bufs[b])
            _scatter_pass(vox_bufs[b], tile_bufs[b], ones16, iota16)
            pltpu.async_copy(tile_bufs[b], _out_slice(j), sems[b])
        return 0

    lax.fori_loop(0, STEPS // NBUF, _pair, 0)

    for b in range(NBUF):
        pltpu.make_async_copy(tile_bufs[b], _out_slice(STEPS - NBUF + b), sems[b]).wait()


def kernel(voxels):
    vox = voxels.reshape(N).astype(jnp.int32)
    mesh = plsc.VectorSubcoreMesh(
        core_axis_name="c",
        subcore_axis_name="s",
        num_cores=NUM_CORES,
        num_subcores=NUM_SUBCORES,
    )
    out = pl.kernel(
        _onehot_body,
        out_type=jax.ShapeDtypeStruct((N, NUM_CLASSES), jnp.float32),
        mesh=mesh,
        scratch_types=[
            pltpu.VMEM((CHUNK,), jnp.int32),
            pltpu.VMEM((CHUNK,), jnp.int32),
            pltpu.VMEM((CHUNK, NUM_CLASSES), jnp.float32),
            pltpu.VMEM((CHUNK, NUM_CLASSES), jnp.float32),
            pltpu.SemaphoreType.DMA,
            pltpu.SemaphoreType.DMA,
        ],
        compiler_params=pltpu.CompilerParams(
            use_tc_tiling_on_sc=False, needs_layout_passes=False
        ),
    )(vox)
    onehot = out.reshape(GRID, GRID, GRID, NUM_CLASSES)
    return jnp.transpose(onehot, (3, 0, 1, 2))


# speedup vs baseline: 6.0170x; 3.7055x over previous
import jax
import jax.numpy as jnp
from jax import lax
from jax.experimental import pallas as pl
from jax.experimental.pallas import tpu as pltpu
from jax.experimental.pallas import tpu_sc as plsc

NUM_CLASSES = 256
GRID = 64
N = GRID * GRID * GRID
NUM_CORES = 2
NUM_SUBCORES = 16
NUM_WORKERS = NUM_CORES * NUM_SUBCORES
LANES = 16

PER_WORKER = N // NUM_WORKERS
CHUNK = 128
STEPS = PER_WORKER // CHUNK
NBUF = 2


def _scatter_pass(vox_v, tile_v, value16, iota16):
    for k in range(CHUNK // LANES):
        vox16 = vox_v[pl.ds(k * LANES, LANES)]
        rows = iota16 + (k * LANES)
        plsc.store_scatter(tile_v, [rows, vox16], value16)


def _onehot_body(vox_hbm, out_hbm, vox0, vox1, tile0, tile1, sem0, sem1):
    cid = lax.axis_index("c")
    sid = lax.axis_index("s")
    wid = sid * NUM_CORES + cid
    row_base = wid * PER_WORKER

    vox_bufs = (vox0, vox1)
    tile_bufs = (tile0, tile1)
    sems = (sem0, sem1)

    zeros16 = jnp.zeros((LANES,), jnp.float32)
    ones16 = jnp.full((LANES,), 1.0, jnp.float32)
    iota16 = lax.iota(jnp.int32, LANES)

    def _zero_row(r, _):
        for b in range(NBUF):
            for k in range(NUM_CLASSES // LANES):
                tile_bufs[b][r, pl.ds(k * LANES, LANES)] = zeros16
        return 0

    lax.fori_loop(0, CHUNK, _zero_row, 0)

    def _out_slice(j):
        off = pl.multiple_of(row_base + j * CHUNK, CHUNK)
        return out_hbm.at[pl.ds(off, CHUNK), :]

    def _pair(t, _):
        for b in range(NBUF):
            j = t * NBUF + b

            @pl.when(t >= 1)
            def _drain():
                pltpu.make_async_copy(tile_bufs[b], _out_slice(j - NBUF), sems[b]).wait()
                _scatter_pass(vox_bufs[b], tile_bufs[b], zeros16, iota16)

            off = pl.multiple_of(row_base + j * CHUNK, CHUNK)
            pltpu.sync_copy(vox_hbm.at[pl.ds(off, CHUNK)], vox_bufs[b])
            _scatter_pass(vox_bufs[b], tile_bufs[b], ones16, iota16)
            pltpu.async_copy(tile_bufs[b], _out_slice(j), sems[b])
        return 0

    lax.fori_loop(0, STEPS // NBUF, _pair, 0)

    for b in range(NBUF):
        pltpu.make_async_copy(tile_bufs[b], _out_slice(STEPS - NBUF + b), sems[b]).wait()


def kernel(voxels):
    vox = voxels.reshape(N).astype(jnp.int32)
    mesh = plsc.VectorSubcoreMesh(
        core_axis_name="c",
        subcore_axis_name="s",
        num_cores=NUM_CORES,
        num_subcores=NUM_SUBCORES,
    )
    out = pl.kernel(
        _onehot_body,
        out_type=jax.ShapeDtypeStruct((N, NUM_CLASSES), jnp.float32),
        mesh=mesh,
        scratch_types=[
            pltpu.VMEM((CHUNK,), jnp.int32),
            pltpu.VMEM((CHUNK,), jnp.int32),
            pltpu.VMEM((CHUNK, NUM_CLASSES), jnp.float32),
            pltpu.VMEM((CHUNK, NUM_CLASSES), jnp.float32),
            pltpu.SemaphoreType.DMA,
            pltpu.SemaphoreType.DMA,
        ],
        compiler_params=pltpu.CompilerParams(
            use_tc_tiling_on_sc=True, needs_layout_passes=False
        ),
    )(vox)
    onehot = out.reshape(GRID, GRID, GRID, NUM_CLASSES)
    return jnp.transpose(onehot, (3, 0, 1, 2))
